# baseline (speedup 1.0000x reference)
import functools

import jax
import jax.numpy as jnp
from jax import lax
from jax.experimental import pallas as pl
from jax.experimental.pallas import tpu as pltpu
from jax.experimental.pallas import tpu_sc as plsc

VOCAB = 100000
D = 768
MAX_LEN = 512
B = 64
S = 512

NC = 2
NS = 16
L = 16
NW = NC * NS

T = 32
CHUNKS = S // T
ROWS_PER_W = B // NW
NSL = D // L

_MESH = plsc.VectorSubcoreMesh(
    core_axis_name="c", subcore_axis_name="s", num_cores=NC, num_subcores=NS
)


def _lanesum(v):
  lanes = lax.iota(jnp.int32, L)
  dnums = lax.GatherDimensionNumbers(
      offset_dims=(), collapsed_slice_dims=(0,), start_index_map=(0,))
  for k in (8, 4, 2, 1):
    idx = (lanes ^ k).reshape(L, 1)
    v = v + lax.gather(v, idx, dnums, (1,),
                       mode=lax.GatherScatterMode.PROMISE_IN_BOUNDS)
  return v


def _rsqrt16(x):
  i = lax.bitcast_convert_type(x, jnp.int32)
  i = jnp.int32(0x5F3759DF) - lax.shift_right_logical(i, 1)
  y = lax.bitcast_convert_type(i, jnp.float32)
  for _ in range(3):
    y = y * (1.5 - 0.5 * x * y * y)
  return y


@functools.partial(
    pl.kernel,
    out_type=jax.ShapeDtypeStruct((B, S, D), jnp.float32),
    mesh=_MESH,
    scratch_types=[
        pltpu.VMEM((T,), jnp.int32),
        pltpu.VMEM((T,), jnp.int32),
        pltpu.VMEM((T, D), jnp.float32),
        pltpu.VMEM((T, D), jnp.float32),
        pltpu.VMEM((T, D), jnp.float32),
        pltpu.VMEM((D,), jnp.float32),
        pltpu.VMEM((D,), jnp.float32),
        pltpu.SemaphoreType.DMA,
        pltpu.SemaphoreType.DMA,
    ],
)
def _embed_ln(x_hbm, seg_hbm, tok_hbm, pos_hbm, segt_hbm, g_hbm, bt_hbm,
              out_hbm, idx_v, sidx_v, tok_v, pos_v, seg_v, gam_v, bet_v,
              sem_t, sem_s):
  wid = lax.axis_index("s") * NC + lax.axis_index("c")
  pltpu.sync_copy(g_hbm, gam_v)
  pltpu.sync_copy(bt_hbm, bet_v)

  def chunk_body(c, _):
    pltpu.sync_copy(pos_hbm.at[pl.ds(c * T, T), :], pos_v)

    def seq_body(s, _):
      b = wid * ROWS_PER_W + s
      pltpu.sync_copy(x_hbm.at[b, pl.ds(c * T, T)], idx_v)
      pltpu.sync_copy(seg_hbm.at[b, pl.ds(c * T, T)], sidx_v)
      cp_t = pltpu.async_copy(tok_hbm.at[idx_v], tok_v, sem_t)
      cp_s = pltpu.async_copy(segt_hbm.at[sidx_v], seg_v, sem_s)
      cp_t.wait()
      cp_s.wait()

      def tok_body(t, _):
        def acc_body(j, carry):
          acc, accsq = carry
          sl = pl.ds(j * L, L)
          v = tok_v[t, sl] + pos_v[t, sl] + seg_v[t, sl]
          tok_v[t, sl] = v
          return acc + v, accsq + v * v

        acc, accsq = lax.fori_loop(
            0, NSL, acc_body,
            (jnp.zeros((L,), jnp.float32), jnp.zeros((L,), jnp.float32)))
        mean = _lanesum(acc) * (1.0 / D)
        ssq = _lanesum(accsq) * (1.0 / D)
        rstd = _rsqrt16(ssq - mean * mean + 1e-5)

        def norm_body(j, _):
          sl = pl.ds(j * L, L)
          v = (tok_v[t, sl] - mean) * rstd
          tok_v[t, sl] = v * gam_v[sl] + bet_v[sl]
          return 0

        lax.fori_loop(0, NSL, norm_body, 0)
        return 0

      lax.fori_loop(0, T, tok_body, 0)
      pltpu.sync_copy(tok_v, out_hbm.at[b, pl.ds(c * T, T), :])
      return 0

    lax.fori_loop(0, ROWS_PER_W, seq_body, 0)
    return 0

  lax.fori_loop(0, CHUNKS, chunk_body, 0)


def kernel(x, seg_ids, tok_table, pos_table, seg_table, gamma, beta):
  return _embed_ln(x, seg_ids, tok_table, pos_table, seg_table, gamma, beta)
